# baseline (speedup 1.0000x reference)
import functools

import jax
import jax.numpy as jnp
from jax import lax
from jax.experimental import pallas as pl
from jax.experimental.pallas import tpu as pltpu
from jax.experimental.pallas import tpu_sc as plsc


def _sc_flatten_idx(B, HIST):
    info = plsc.get_sparse_core_info()
    nc, ns = info.num_cores, info.num_subcores
    nw = nc * ns
    assert B % nw == 0
    bpw = B // nw
    n = bpw * HIST
    offs = sorted({min(16 * k, HIST - 16) for k in range((HIST + 15) // 16)})

    mesh = plsc.VectorSubcoreMesh(core_axis_name="c", subcore_axis_name="s")

    @functools.partial(
        pl.kernel,
        mesh=mesh,
        out_type=jax.ShapeDtypeStruct((B * HIST,), jnp.int32),
        scratch_types=[
            pltpu.VMEM((bpw, HIST), jnp.int32),
            pltpu.VMEM((n,), jnp.int32),
        ],
    )
    def sc_flat(idx_hbm, out_hbm, v2, flat):
        wid = lax.axis_index("s") * nc + lax.axis_index("c")
        pltpu.sync_copy(idx_hbm.at[pl.ds(wid * bpw, bpw)], v2)

        def body(j, carry):
            for o in offs:
                flat[pl.ds(j * HIST + o, 16)] = v2[j, pl.ds(o, 16)]
            return carry

        lax.fori_loop(0, bpw, body, 0)
        pltpu.sync_copy(flat, out_hbm.at[pl.ds(wid * n, n)])

    return sc_flat


def _sc_transpose_table(V, D):
    info = plsc.get_sparse_core_info()
    nc, ns = info.num_cores, info.num_subcores
    nw = nc * ns
    assert D == 64
    K = 4
    CV = 128 * K
    full_tiles = V // 128
    rem = V % 128
    tpw = full_tiles // nw
    assert tpw % K == 0
    cpw = tpw // K
    extra = full_tiles - tpw * nw
    nd = D // 8

    mesh = plsc.VectorSubcoreMesh(core_axis_name="c", subcore_axis_name="s")

    @functools.partial(
        pl.kernel,
        mesh=mesh,
        compiler_params=pltpu.CompilerParams(needs_layout_passes=False),
        out_type=jax.ShapeDtypeStruct((V * D,), jnp.float32),
        scratch_types=[
            pltpu.VMEM((D, CV + 1), jnp.float32),
            pltpu.VMEM((CV * D,), jnp.float32),
            pltpu.VMEM((max(rem, 1) * D,), jnp.float32),
            pltpu.SemaphoreType.DMA,
        ],
    )
    def sc_tr(tin_hbm, tail_hbm, out_hbm, in_v, out_v, tail_v, sem):
        wid = lax.axis_index("s") * nc + lax.axis_index("c")
        d_idx = [lax.iota(jnp.int32, 16) + 16 * k for k in range(D // 16)]

        def do_chunk(v0, ncols, nv):
            cps = [pltpu.make_async_copy(
                       tin_hbm.at[pl.ds(8 * db, 8), pl.ds(v0, ncols)],
                       in_v.at[pl.ds(8 * db, 8), pl.ds(0, ncols)], sem)
                   for db in range(nd)]
            for c in cps:
                c.start()
            for c in cps:
                c.wait()

            def body(vh, carry):
                for u in range(2):
                    v = vh * 2 + u
                    vv = jnp.full((16,), v, jnp.int32)
                    for k in range(D // 16):
                        val = plsc.load_gather(in_v, [d_idx[k], vv])
                        out_v[pl.ds(v * D + 16 * k, 16)] = val
                return carry

            lax.fori_loop(0, nv // 2, body, 0)
            pltpu.sync_copy(out_v.at[pl.ds(0, ncols * D)],
                            out_hbm.at[pl.ds(v0 * D, ncols * D)])

        def main(c, carry):
            v0 = pl.multiple_of((wid * tpw + c * K) * 128, 8)
            do_chunk(v0, CV, CV)
            return carry

        lax.fori_loop(0, cpw, main, 0)

        for e in range(extra):
            @pl.when(wid == e)
            def _(e=e):
                do_chunk((tpw * nw + e) * 128, 128, 128)
        if rem:
            @pl.when(wid == extra)
            def _():
                pltpu.sync_copy(tail_hbm, tail_v)
                pltpu.sync_copy(
                    tail_v, out_hbm.at[pl.ds(full_tiles * 128 * D, rem * D)])

    return sc_tr


def _sc_gather_sum(B, HIST, D, ROWW):
    info = plsc.get_sparse_core_info()
    nc, ns = info.num_cores, info.num_subcores
    nw = nc * ns
    assert B % nw == 0
    bpw = B // nw

    n_vec = D // 16

    G = 2
    NBUF = 2
    GH = G * HIST
    NG = bpw // G
    U = 8
    assert bpw % (G * NBUF) == 0 and HIST % U == 0 and GH % 8 == 0
    chunks = [(o, min(128, GH - o)) for o in range(0, GH, 128)]
    assert all(o % 8 == 0 for o, _ in chunks)

    mesh = plsc.VectorSubcoreMesh(core_axis_name="c", subcore_axis_name="s")

    @functools.partial(
        pl.kernel,
        mesh=mesh,
        compiler_params=pltpu.CompilerParams(use_tc_tiling_on_sc=False),
        out_type=jax.ShapeDtypeStruct((B, D), jnp.float32),
        scratch_types=[
            pltpu.VMEM((bpw * HIST,), jnp.int32),
            pltpu.VMEM((NBUF, GH, ROWW), jnp.float32),
            pltpu.VMEM((bpw, D), jnp.float32),
        ] + [pltpu.SemaphoreType.DMA] * NBUF,
    )
    def sc_sum(idx_hbm, table_hbm, out_hbm, idx_v, rows_v, stage_v, *sems):
        wid = lax.axis_index("s") * nc + lax.axis_index("c")
        base = wid * bpw
        pltpu.sync_copy(idx_hbm.at[pl.ds(base * HIST, bpw * HIST)], idx_v)

        def _copies(g, buf):
            off = pl.multiple_of(g * GH, 8)
            return [pltpu.make_async_copy(
                        table_hbm.at[idx_v.at[pl.ds(off + o, l)]],
                        rows_v.at[buf, pl.ds(o, l)],
                        sems[buf])
                    for o, l in chunks]

        def issue(g, buf):
            for c in _copies(g, buf):
                c.start()

        def drain(g, buf):
            for c in _copies(g, buf):
                c.wait()

        zero = jnp.zeros((16,), jnp.float32)

        def accum(g, buf):
            for rr in range(G):
                def body(jj, accs, _rr=rr):
                    j0 = _rr * HIST + jj * U
                    for u in range(U):
                        accs = tuple(
                            accs[k] + rows_v[buf, j0 + u, pl.ds(16 * k, 16)]
                            for k in range(n_vec))
                    return accs

                accs = lax.fori_loop(0, HIST // U, body, (zero,) * n_vec)
                r_out = g * G + rr
                for k in range(n_vec):
                    stage_v[r_out, pl.ds(16 * k, 16)] = accs[k]

        issue(0, 0)

        def outer(i, carry):
            g0 = i * NBUF
            for b in range(NBUF):
                cur = g0 + b
                nxt = cur + 1

                @pl.when(nxt < NG)
                def _(nxt=nxt, b=b):
                    issue(nxt, (b + 1) % NBUF)

                drain(cur, b)
                accum(cur, b)
            return carry

        lax.fori_loop(0, NG // NBUF, outer, 0)
        pltpu.sync_copy(stage_v, out_hbm.at[pl.ds(base, bpw)])

    return sc_sum


def _tc_linear(sums, W, b2, scale):
    B, D = sums.shape
    OUT = W.shape[1]
    blk = 512 if B % 512 == 0 else B

    def body(s_ref, w_ref, b_ref, o_ref):
        o_ref[...] = jnp.dot(s_ref[...] * scale, w_ref[...],
                             preferred_element_type=jnp.float32) + b_ref[...]

    return pl.pallas_call(
        body,
        grid=(B // blk,),
        in_specs=[
            pl.BlockSpec((blk, D), lambda i: (i, 0)),
            pl.BlockSpec((D, OUT), lambda i: (0, 0)),
            pl.BlockSpec((1, OUT), lambda i: (0, 0)),
        ],
        out_specs=pl.BlockSpec((blk, OUT), lambda i: (i, 0)),
        out_shape=jax.ShapeDtypeStruct((B, OUT), jnp.float32),
    )(sums, W, b2)


def kernel(word_indices, embedding, W, b):
    B, HIST = word_indices.shape
    D = embedding.shape[1]
    V = embedding.shape[0]
    idx_flat = _sc_flatten_idx(B, HIST)(word_indices.astype(jnp.int32))
    tail = embedding[(V // 128) * 128:, :].reshape(-1)
    emb_lin = _sc_transpose_table(V, D)(jnp.transpose(embedding), tail)
    sums = _sc_gather_sum(B, HIST, D, D)(idx_flat, emb_lin.reshape(V, D))
    return _tc_linear(sums, W, b.reshape(1, -1), 1.0 / HIST)


# speedup vs baseline: 2.5867x; 2.5867x over previous
import functools

import jax
import jax.numpy as jnp
from jax import lax
from jax.experimental import pallas as pl
from jax.experimental.pallas import tpu as pltpu
from jax.experimental.pallas import tpu_sc as plsc


def _sc_flatten_idx(B, HIST):
    info = plsc.get_sparse_core_info()
    nc, ns = info.num_cores, info.num_subcores
    nw = nc * ns
    assert B % nw == 0
    bpw = B // nw
    n = bpw * HIST
    offs = sorted({min(16 * k, HIST - 16) for k in range((HIST + 15) // 16)})

    mesh = plsc.VectorSubcoreMesh(core_axis_name="c", subcore_axis_name="s")

    @functools.partial(
        pl.kernel,
        mesh=mesh,
        out_type=jax.ShapeDtypeStruct((B * HIST,), jnp.int32),
        scratch_types=[
            pltpu.VMEM((bpw, HIST), jnp.int32),
            pltpu.VMEM((n,), jnp.int32),
        ],
    )
    def sc_flat(idx_hbm, out_hbm, v2, flat):
        wid = lax.axis_index("s") * nc + lax.axis_index("c")
        pltpu.sync_copy(idx_hbm.at[pl.ds(wid * bpw, bpw)], v2)

        def body(j, carry):
            for o in offs:
                flat[pl.ds(j * HIST + o, 16)] = v2[j, pl.ds(o, 16)]
            return carry

        lax.fori_loop(0, bpw, body, 0)
        pltpu.sync_copy(flat, out_hbm.at[pl.ds(wid * n, n)])

    return sc_flat


def _sc_gather_sum(B, HIST, D, ROWW):
    info = plsc.get_sparse_core_info()
    nc, ns = info.num_cores, info.num_subcores
    nw = nc * ns
    assert B % nw == 0
    bpw = B // nw

    n_vec = D // 16

    G = 2
    NBUF = 2
    GH = G * HIST
    NG = bpw // G
    U = 8
    assert bpw % (G * NBUF) == 0 and HIST % U == 0 and GH % 8 == 0
    chunks = [(o, min(128, GH - o)) for o in range(0, GH, 128)]
    assert all(o % 8 == 0 for o, _ in chunks)

    mesh = plsc.VectorSubcoreMesh(core_axis_name="c", subcore_axis_name="s")

    @functools.partial(
        pl.kernel,
        mesh=mesh,
        compiler_params=pltpu.CompilerParams(use_tc_tiling_on_sc=False),
        out_type=jax.ShapeDtypeStruct((B, D), jnp.float32),
        scratch_types=[
            pltpu.VMEM((bpw * HIST,), jnp.int32),
            pltpu.VMEM((NBUF, GH, ROWW), jnp.float32),
            pltpu.VMEM((bpw, D), jnp.float32),
        ] + [pltpu.SemaphoreType.DMA] * NBUF,
    )
    def sc_sum(idx_hbm, table_hbm, out_hbm, idx_v, rows_v, stage_v, *sems):
        wid = lax.axis_index("s") * nc + lax.axis_index("c")
        base = wid * bpw
        pltpu.sync_copy(idx_hbm.at[pl.ds(base * HIST, bpw * HIST)], idx_v)

        def _copies(g, buf):
            off = pl.multiple_of(g * GH, 8)
            return [pltpu.make_async_copy(
                        table_hbm.at[idx_v.at[pl.ds(off + o, l)]],
                        rows_v.at[buf, pl.ds(o, l)],
                        sems[buf])
                    for o, l in chunks]

        def issue(g, buf):
            for c in _copies(g, buf):
                c.start()

        def drain(g, buf):
            for c in _copies(g, buf):
                c.wait()

        zero = jnp.zeros((16,), jnp.float32)

        def accum(g, buf):
            for rr in range(G):
                def body(jj, accs, _rr=rr):
                    j0 = _rr * HIST + jj * U
                    for u in range(U):
                        accs = tuple(
                            accs[k] + rows_v[buf, j0 + u, pl.ds(16 * k, 16)]
                            for k in range(n_vec))
                    return accs

                accs = lax.fori_loop(0, HIST // U, body, (zero,) * n_vec)
                r_out = g * G + rr
                for k in range(n_vec):
                    stage_v[r_out, pl.ds(16 * k, 16)] = accs[k]

        issue(0, 0)

        def outer(i, carry):
            g0 = i * NBUF
            for b in range(NBUF):
                cur = g0 + b
                nxt = cur + 1

                @pl.when(nxt < NG)
                def _(nxt=nxt, b=b):
                    issue(nxt, (b + 1) % NBUF)

                drain(cur, b)
                accum(cur, b)
            return carry

        lax.fori_loop(0, NG // NBUF, outer, 0)
        pltpu.sync_copy(stage_v, out_hbm.at[pl.ds(base, bpw)])

    return sc_sum


def _tc_linear(sums, W, b2, scale):
    B, D = sums.shape
    OUT = W.shape[1]
    blk = 512 if B % 512 == 0 else B

    def body(s_ref, w_ref, b_ref, o_ref):
        o_ref[...] = jnp.dot(s_ref[...] * scale, w_ref[...],
                             preferred_element_type=jnp.float32) + b_ref[...]

    return pl.pallas_call(
        body,
        grid=(B // blk,),
        in_specs=[
            pl.BlockSpec((blk, D), lambda i: (i, 0)),
            pl.BlockSpec((D, OUT), lambda i: (0, 0)),
            pl.BlockSpec((1, OUT), lambda i: (0, 0)),
        ],
        out_specs=pl.BlockSpec((blk, OUT), lambda i: (i, 0)),
        out_shape=jax.ShapeDtypeStruct((B, OUT), jnp.float32),
    )(sums, W, b2)


def kernel(word_indices, embedding, W, b):
    B, HIST = word_indices.shape
    D = embedding.shape[1]
    idx_flat = _sc_flatten_idx(B, HIST)(word_indices.astype(jnp.int32))
    sums = _sc_gather_sum(B, HIST, D, D)(idx_flat, embedding)
    return _tc_linear(sums, W, b.reshape(1, -1), 1.0 / HIST)


# speedup vs baseline: 2.6518x; 1.0252x over previous
import functools

import jax
import jax.numpy as jnp
from jax import lax
from jax.experimental import pallas as pl
from jax.experimental.pallas import tpu as pltpu
from jax.experimental.pallas import tpu_sc as plsc


def _sc_flatten_idx(B, HIST):
    info = plsc.get_sparse_core_info()
    nc, ns = info.num_cores, info.num_subcores
    nw = nc * ns
    assert B % nw == 0
    bpw = B // nw
    n = bpw * HIST
    offs = sorted({min(16 * k, HIST - 16) for k in range((HIST + 15) // 16)})

    mesh = plsc.VectorSubcoreMesh(core_axis_name="c", subcore_axis_name="s")

    @functools.partial(
        pl.kernel,
        mesh=mesh,
        out_type=jax.ShapeDtypeStruct((B * HIST,), jnp.int32),
        scratch_types=[
            pltpu.VMEM((bpw, HIST), jnp.int32),
            pltpu.VMEM((n,), jnp.int32),
        ],
    )
    def sc_flat(idx_hbm, out_hbm, v2, flat):
        wid = lax.axis_index("s") * nc + lax.axis_index("c")
        pltpu.sync_copy(idx_hbm.at[pl.ds(wid * bpw, bpw)], v2)

        def body(j, carry):
            for o in offs:
                flat[pl.ds(j * HIST + o, 16)] = v2[j, pl.ds(o, 16)]
            return carry

        lax.fori_loop(0, bpw, body, 0)
        pltpu.sync_copy(flat, out_hbm.at[pl.ds(wid * n, n)])

    return sc_flat


def _sc_gather_sum(B, HIST, D, ROWW):
    info = plsc.get_sparse_core_info()
    nc, ns = info.num_cores, info.num_subcores
    nw = nc * ns
    assert B % nw == 0
    bpw = B // nw

    n_vec = D // 16

    G = 1
    NBUF = 4
    GH = G * HIST
    NG = bpw // G
    U = 8
    assert bpw % (G * NBUF) == 0 and HIST % U == 0 and GH % 8 == 0
    chunks = [(o, min(128, GH - o)) for o in range(0, GH, 128)]
    assert all(o % 8 == 0 for o, _ in chunks)

    mesh = plsc.VectorSubcoreMesh(core_axis_name="c", subcore_axis_name="s")

    @functools.partial(
        pl.kernel,
        mesh=mesh,
        compiler_params=pltpu.CompilerParams(use_tc_tiling_on_sc=False),
        out_type=jax.ShapeDtypeStruct((B, D), jnp.float32),
        scratch_types=[
            pltpu.VMEM((bpw * HIST,), jnp.int32),
            pltpu.VMEM((NBUF, GH, ROWW), jnp.float32),
            pltpu.VMEM((bpw, D), jnp.float32),
        ] + [pltpu.SemaphoreType.DMA] * NBUF,
    )
    def sc_sum(idx_hbm, table_hbm, out_hbm, idx_v, rows_v, stage_v, *sems):
        wid = lax.axis_index("s") * nc + lax.axis_index("c")
        base = wid * bpw
        pltpu.sync_copy(idx_hbm.at[pl.ds(base * HIST, bpw * HIST)], idx_v)

        def _copies(g, buf):
            off = pl.multiple_of(g * GH, 8)
            return [pltpu.make_async_copy(
                        table_hbm.at[idx_v.at[pl.ds(off + o, l)]],
                        rows_v.at[buf, pl.ds(o, l)],
                        sems[buf])
                    for o, l in chunks]

        def issue(g, buf):
            for c in _copies(g, buf):
                c.start()

        def drain(g, buf):
            for c in _copies(g, buf):
                c.wait()

        zero = jnp.zeros((16,), jnp.float32)

        def accum(g, buf):
            for rr in range(G):
                def body(jj, accs, _rr=rr):
                    j0 = _rr * HIST + jj * U
                    for u in range(U):
                        accs = tuple(
                            accs[k] + rows_v[buf, j0 + u, pl.ds(16 * k, 16)]
                            for k in range(n_vec))
                    return accs

                accs = lax.fori_loop(0, HIST // U, body, (zero,) * n_vec)
                r_out = g * G + rr
                for k in range(n_vec):
                    stage_v[r_out, pl.ds(16 * k, 16)] = accs[k]

        for p in range(NBUF - 1):
            issue(p, p)

        def outer(i, carry):
            g0 = i * NBUF
            for b in range(NBUF):
                cur = g0 + b
                nxt = cur + NBUF - 1

                @pl.when(nxt < NG)
                def _(nxt=nxt, b=b):
                    issue(nxt, (b + NBUF - 1) % NBUF)

                drain(cur, b)
                accum(cur, b)
            return carry

        lax.fori_loop(0, NG // NBUF, outer, 0)
        pltpu.sync_copy(stage_v, out_hbm.at[pl.ds(base, bpw)])

    return sc_sum


def _tc_linear(sums, W, b2, scale):
    B, D = sums.shape
    OUT = W.shape[1]
    blk = 512 if B % 512 == 0 else B

    def body(s_ref, w_ref, b_ref, o_ref):
        o_ref[...] = jnp.dot(s_ref[...] * scale, w_ref[...],
                             preferred_element_type=jnp.float32) + b_ref[...]

    return pl.pallas_call(
        body,
        grid=(B // blk,),
        in_specs=[
            pl.BlockSpec((blk, D), lambda i: (i, 0)),
            pl.BlockSpec((D, OUT), lambda i: (0, 0)),
            pl.BlockSpec((1, OUT), lambda i: (0, 0)),
        ],
        out_specs=pl.BlockSpec((blk, OUT), lambda i: (i, 0)),
        out_shape=jax.ShapeDtypeStruct((B, OUT), jnp.float32),
    )(sums, W, b2)


def kernel(word_indices, embedding, W, b):
    B, HIST = word_indices.shape
    D = embedding.shape[1]
    idx_flat = _sc_flatten_idx(B, HIST)(word_indices.astype(jnp.int32))
    sums = _sc_gather_sum(B, HIST, D, D)(idx_flat, embedding)
    return _tc_linear(sums, W, b.reshape(1, -1), 1.0 / HIST)
